# baseline (speedup 1.0000x reference)
import functools

import jax
import jax.numpy as jnp
from jax import lax
from jax.experimental import pallas as pl
from jax.experimental.pallas import tpu as pltpu
from jax.experimental.pallas import tpu_sc as plsc

NBUF = 4
LANES = 16
PACK = 4
CH = 256
D = 32


def _sc_geometry():
    try:
        info = plsc.get_sparse_core_info()
        return info.num_cores, info.num_subcores
    except Exception:
        return 2, 16


def _bcast_lane(vec, lane):
    idx = jnp.full((LANES,), lane, jnp.int32)
    return lax.gather(
        vec,
        idx[:, None],
        lax.GatherDimensionNumbers(
            offset_dims=(),
            collapsed_slice_dims=(0,),
            start_index_map=(0,),
        ),
        slice_sizes=(1,),
        mode=lax.GatherScatterMode.PROMISE_IN_BOUNDS,
    )


def _make_repack(V, NC, NS):
    NW = NC * NS
    VT = V - V % 128
    TAIL = V - VT
    NCHUNK = VT // CH
    CPW = (NCHUNK // NW) & ~1
    NEXTRA = NCHUNK - CPW * NW
    assert NEXTRA <= NW
    OROW = CH // PACK

    mesh = plsc.VectorSubcoreMesh(core_axis_name="c", subcore_axis_name="s")

    @functools.partial(
        pl.kernel,
        out_type=jax.ShapeDtypeStruct((V // PACK, PACK * D), jnp.float32),
        mesh=mesh,
        scratch_types=[
            pltpu.VMEM((2, D, CH), jnp.float32),
            pltpu.VMEM((D, CH + 512), jnp.float32),
            pltpu.VMEM((2, OROW, PACK * D), jnp.float32),
            pltpu.VMEM((TAIL, D), jnp.float32),
            pltpu.VMEM((TAIL // PACK, PACK * D), jnp.float32),
        ] + [pltpu.SemaphoreType.DMA] * 4,
        compiler_params=pltpu.CompilerParams(
            use_tc_tiling_on_sc=True, needs_layout_passes=False
        ),
    )
    def repack(tt_hbm, tail_hbm, out_hbm, in_v, skew_v, row_v, tail_v,
               trow_v, *sems):
        wid = lax.axis_index("s") * NC + lax.axis_index("c")
        iota = lax.iota(jnp.int32, LANES)

        def chunk_id(k):
            return k * NW + wid

        def start_in(k, rb):
            pltpu.async_copy(
                tt_hbm.at[:, pl.ds(chunk_id(k) * CH, CH)], in_v.at[rb],
                sems[rb],
            )

        def wait_in(k, rb):
            pltpu.make_async_copy(
                tt_hbm.at[:, pl.ds(chunk_id(k) * CH, CH)], in_v.at[rb],
                sems[rb],
            ).wait()

        def start_out(k, rb):
            pltpu.async_copy(
                row_v.at[rb], out_hbm.at[pl.ds(chunk_id(k) * OROW, OROW)],
                sems[2 + rb],
            )

        def wait_out(k, rb):
            pltpu.make_async_copy(
                row_v.at[rb], out_hbm.at[pl.ds(chunk_id(k) * OROW, OROW)],
                sems[2 + rb],
            ).wait()

        def transpose_panel(src, dst, n):
            GRP = 4 * PACK

            def prow(q, carry):
                gs = []
                for u in range(GRP):
                    wv = jnp.broadcast_to(q * GRP + u, (LANES,)).astype(
                        jnp.int32
                    )
                    gs.append(plsc.load_gather(src, [iota, wv]))
                    gs.append(
                        plsc.load_gather(src, [iota + jnp.int32(LANES), wv])
                    )
                for u in range(GRP):
                    p = (GRP // PACK) * q + u // PACK
                    dst[p, pl.ds((u % PACK) * D, LANES)] = gs[2 * u]
                    dst[p, pl.ds((u % PACK) * D + LANES, LANES)] = gs[2 * u + 1]
                return carry

            lax.fori_loop(0, n // GRP, prow, jnp.int32(0))

        start_in(jnp.int32(0), 0)
        start_in(jnp.int32(1), 1)

        def body(q, carry):
            for rb in range(2):
                k = 2 * q + rb
                wait_in(k, rb)

                @pl.when(k >= 2)
                def _():
                    wait_out(k - 2, rb)

                transpose_panel(in_v.at[rb], row_v.at[rb], CH)
                start_out(k, rb)

                @pl.when(k + 2 < CPW)
                def _():
                    start_in(k + 2, rb)

            return carry

        lax.fori_loop(0, CPW // 2, body, jnp.int32(0))
        wait_out(jnp.int32(CPW - 2), 0)
        wait_out(jnp.int32(CPW - 1), 1)

        @pl.when(wid < NEXTRA)
        def _():
            c = jnp.int32(CPW * NW) + wid
            pltpu.sync_copy(tt_hbm.at[:, pl.ds(c * CH, CH)], in_v.at[0])
            transpose_panel(in_v.at[0], row_v.at[0], CH)
            pltpu.sync_copy(row_v.at[0], out_hbm.at[pl.ds(c * OROW, OROW)])

        def transpose_tail():
            def prow(p, carry):
                for u in range(PACK):
                    tv = jnp.broadcast_to(p * PACK + u, (LANES,)).astype(
                        jnp.int32
                    )
                    g0 = plsc.load_gather(tail_v, [tv, iota])
                    g1 = plsc.load_gather(tail_v, [tv, iota + jnp.int32(LANES)])
                    trow_v[p, pl.ds(u * D, LANES)] = g0
                    trow_v[p, pl.ds(u * D + LANES, LANES)] = g1
                return carry

            lax.fori_loop(0, TAIL // PACK, prow, jnp.int32(0))

        @pl.when(wid == 0)
        def _():
            pltpu.sync_copy(tail_hbm, tail_v)
            transpose_tail()
            pltpu.sync_copy(
                trow_v, out_hbm.at[pl.ds(VT // PACK, TAIL // PACK)]
            )

    return repack


def _make_encoder(B, L, NC, NS):
    NW = NC * NS
    assert B % NW == 0
    BPW = B // NW
    assert BPW % NBUF == 0
    RPAD = 64
    inv_l = jnp.float32(1.0 / L)

    mesh = plsc.VectorSubcoreMesh(core_axis_name="c", subcore_axis_name="s")

    @functools.partial(
        pl.kernel,
        out_type=jax.ShapeDtypeStruct((B, D), jnp.float32),
        mesh=mesh,
        scratch_types=[
            pltpu.VMEM((BPW, L), jnp.int32),
            pltpu.VMEM((BPW, RPAD), jnp.int32),
            pltpu.VMEM((NBUF, L, PACK * D), jnp.float32),
            pltpu.VMEM((BPW, D), jnp.float32),
        ] + [pltpu.SemaphoreType.DMA] * NBUF,
        compiler_params=pltpu.CompilerParams(
            use_tc_tiling_on_sc=True, needs_layout_passes=False
        ),
    )
    def enc(idx_hbm, off_hbm, table_hbm, out_hbm, idx_v, off_v, rows_v,
            out_v, *sems):
        wid = lax.axis_index("s") * NC + lax.axis_index("c")
        base = wid * BPW

        pltpu.sync_copy(idx_hbm.at[pl.ds(base, BPW)], idx_v)
        pltpu.sync_copy(off_hbm.at[pl.ds(base, BPW)], off_v)

        def start(i, b):
            pltpu.async_copy(table_hbm.at[idx_v.at[i]], rows_v.at[b], sems[b])

        def wait(i, b):
            pltpu.make_async_copy(
                table_hbm.at[idx_v.at[i]], rows_v.at[b], sems[b]
            ).wait()

        for b in range(NBUF):
            start(jnp.int32(b), b)

        iota = lax.iota(jnp.int32, LANES)

        def group(g, carry):
            for b in range(NBUF):
                i = g * NBUF + b
                wait(i, b)
                acc0 = jnp.zeros((LANES,), jnp.float32)
                acc1 = jnp.zeros((LANES,), jnp.float32)
                for j in range(L):
                    ovec = off_v[i, pl.ds((j // LANES) * LANES, LANES)]
                    os_ = _bcast_lane(ovec, j % LANES)
                    js = jnp.full((LANES,), j, jnp.int32)
                    a0 = os_ + iota
                    acc0 = acc0 + plsc.load_gather(rows_v.at[b], [js, a0])
                    acc1 = acc1 + plsc.load_gather(
                        rows_v.at[b], [js, a0 + jnp.int32(LANES)]
                    )
                out_v[i, pl.ds(0, LANES)] = acc0 * inv_l
                out_v[i, pl.ds(LANES, LANES)] = acc1 * inv_l

                @pl.when(g < BPW // NBUF - 1)
                def _():
                    start(i + jnp.int32(NBUF), b)

            return carry

        lax.fori_loop(0, BPW // NBUF, group, jnp.int32(0))

        pltpu.sync_copy(out_v, out_hbm.at[pl.ds(base, BPW)])

    return enc


def kernel(captions, cap_features, w2v_table):
    del cap_features
    B, L = captions.shape
    V, d = w2v_table.shape
    assert d == D and V % PACK == 0
    NC, NS = _sc_geometry()
    VT = V - V % 128
    tableT = w2v_table.T
    tail = w2v_table[VT:, :]
    packed = _make_repack(V, NC, NS)(tableT, tail)
    cap = captions.astype(jnp.int32)
    idx_p = cap >> 2
    off = jnp.pad((cap & 3) * D, ((0, 0), (0, 64 - L)))
    enc = _make_encoder(B, L, NC, NS)
    return enc(idx_p, off, packed)


# speedup vs baseline: 1.1056x; 1.1056x over previous
import functools

import jax
import jax.numpy as jnp
from jax import lax
from jax.experimental import pallas as pl
from jax.experimental.pallas import tpu as pltpu
from jax.experimental.pallas import tpu_sc as plsc

NBUF = 4
LANES = 16


def _sc_geometry():
    try:
        info = plsc.get_sparse_core_info()
        return info.num_cores, info.num_subcores
    except Exception:
        return 2, 16


def _make_encoder(B, L, NC, NS):
    NW = NC * NS
    assert B % NW == 0
    BPW = B // NW
    assert BPW % NBUF == 0
    G = BPW // NBUF
    L2 = 2 * L
    inv_l = jnp.float32(1.0 / L)

    mesh = plsc.VectorSubcoreMesh(core_axis_name="c", subcore_axis_name="s")

    @functools.partial(
        pl.kernel,
        out_type=jax.ShapeDtypeStruct((B, 2 * LANES), jnp.float32),
        mesh=mesh,
        scratch_types=[
            pltpu.VMEM((BPW, L2), jnp.int32),
            pltpu.VMEM((NBUF, L2, LANES), jnp.float32),
            pltpu.VMEM((BPW, 2 * LANES), jnp.float32),
        ] + [pltpu.SemaphoreType.DMA] * NBUF,
        compiler_params=pltpu.CompilerParams(use_tc_tiling_on_sc=False),
    )
    def enc(idx_hbm, table_hbm, out_hbm, idx_v, rows_v, out_v, *sems):
        wid = lax.axis_index("s") * NC + lax.axis_index("c")
        base = wid * BPW

        pltpu.sync_copy(idx_hbm.at[pl.ds(base, BPW)], idx_v)

        def start(i, b):
            pltpu.async_copy(table_hbm.at[idx_v.at[i]], rows_v.at[b], sems[b])

        def wait(i, b):
            pltpu.make_async_copy(
                table_hbm.at[idx_v.at[i]], rows_v.at[b], sems[b]
            ).wait()

        for b in range(NBUF):
            start(jnp.int32(b), b)

        def group(g, carry):
            for b in range(NBUF):
                i = g * NBUF + b
                wait(i, b)
                acc0 = jnp.zeros((LANES,), jnp.float32)
                acc1 = jnp.zeros((LANES,), jnp.float32)
                for r in range(L):
                    acc0 = acc0 + rows_v[b, 2 * r]
                    acc1 = acc1 + rows_v[b, 2 * r + 1]
                out_v[i, pl.ds(0, LANES)] = acc0 * inv_l
                out_v[i, pl.ds(LANES, LANES)] = acc1 * inv_l

                @pl.when(g < G - 1)
                def _():
                    start(i + NBUF, b)

            return carry

        lax.fori_loop(0, G, group, jnp.int32(0))

        pltpu.sync_copy(out_v, out_hbm.at[pl.ds(base, BPW)])

    return enc


def kernel(captions, cap_features, w2v_table):
    del cap_features
    B, L = captions.shape
    V, D = w2v_table.shape
    assert D == 2 * LANES
    NC, NS = _sc_geometry()
    table2 = w2v_table.reshape(2 * V, LANES)
    idx2 = (
        captions[:, :, None].astype(jnp.int32) * 2
        + jnp.arange(2, dtype=jnp.int32)
    ).reshape(B, 2 * L)
    enc = _make_encoder(B, L, NC, NS)
    return enc(idx2, table2)
